# baseline (speedup 1.0000x reference)
import functools

import jax
import jax.numpy as jnp
from jax import lax
from jax.experimental import pallas as pl
from jax.experimental.pallas import tpu as pltpu
from jax.experimental.pallas import tpu_sc as plsc

B = 4096
L = 200
D = 64
N = B * L
NC = 2
NS = 16
NW = NC * NS
BG = B // NW
CH_L = 2
N_CHUNKS = L // CH_L
LANES = 16


def _body(idx_hbm, tok_hbm, pos_hbm, out_hbm,
          idx_a, idx_b, rows_a0, rows_a1, rows_b0, rows_b1, out_a, out_b,
          pos_v, gsem_a, gsem_b, osem_a, osem_b):
    cid = lax.axis_index("c")
    sid = lax.axis_index("s")
    wid = sid * NC + cid

    pltpu.sync_copy(pos_hbm, pos_v)

    def fire(c, idx_v, rows0, rows1, gsem):
        lt = c >> 2
        ll0 = (c & 3) * CH_L
        pltpu.sync_copy(idx_hbm.at[lt, wid, pl.ds(ll0, CH_L)], idx_v)
        pltpu.async_copy(tok_hbm.at[idx_v.at[0]], rows0, gsem)
        pltpu.async_copy(tok_hbm.at[idx_v.at[1]], rows1, gsem)

    def gdrain(idx_v, rows0, rows1, gsem):
        pltpu.make_async_copy(tok_hbm.at[idx_v.at[0]], rows0, gsem).wait()
        pltpu.make_async_copy(tok_hbm.at[idx_v.at[1]], rows1, gsem).wait()

    def repack(rows0, rows1, c, out_v):
        iota = lax.iota(jnp.int32, LANES)

        def d_body(d, carry):
            td = d >> 3
            dd = d & 7
            col = jnp.full((LANES,), d, jnp.int32)
            for p, rows_v in ((0, rows0), (1, rows1)):
                lvec = jnp.full((LANES,), c * CH_L + p, jnp.int32)
                pv = plsc.load_gather(pos_v, [lvec, col])
                for bb in range(BG // LANES):
                    row = iota + (bb * LANES)
                    g = plsc.load_gather(rows_v, [row, col])
                    out_v[p, td, dd, pl.ds(bb * LANES, LANES)] = g + pv
            return carry

        lax.fori_loop(0, D, d_body, 0)

    def ofire(c, out_v, osem):
        pltpu.async_copy(
            out_v, out_hbm.at[pl.ds(c * CH_L, CH_L), pl.ds(0, 8), wid], osem
        )

    def odrain(c, out_v, osem):
        pltpu.make_async_copy(
            out_v, out_hbm.at[pl.ds(c * CH_L, CH_L), pl.ds(0, 8), wid], osem
        ).wait()

    fire(0, idx_a, rows_a0, rows_a1, gsem_a)
    fire(1, idx_b, rows_b0, rows_b1, gsem_b)
    gdrain(idx_a, rows_a0, rows_a1, gsem_a)
    repack(rows_a0, rows_a1, 0, out_a)
    ofire(0, out_a, osem_a)
    fire(2, idx_a, rows_a0, rows_a1, gsem_a)
    gdrain(idx_b, rows_b0, rows_b1, gsem_b)
    repack(rows_b0, rows_b1, 1, out_b)
    ofire(1, out_b, osem_b)
    fire(3, idx_b, rows_b0, rows_b1, gsem_b)

    def pair_body(cc, carry):
        c0 = 2 * cc
        gdrain(idx_a, rows_a0, rows_a1, gsem_a)
        odrain(c0, out_a, osem_a)
        repack(rows_a0, rows_a1, c0, out_a)
        ofire(c0, out_a, osem_a)
        fire(c0 + 2, idx_a, rows_a0, rows_a1, gsem_a)
        gdrain(idx_b, rows_b0, rows_b1, gsem_b)
        odrain(c0 + 1, out_b, osem_b)
        repack(rows_b0, rows_b1, c0 + 1, out_b)
        ofire(c0 + 1, out_b, osem_b)
        fire(c0 + 3, idx_b, rows_b0, rows_b1, gsem_b)
        return carry

    lax.fori_loop(1, N_CHUNKS // 2 - 1, pair_body, 0)

    c_last = N_CHUNKS - 2
    gdrain(idx_a, rows_a0, rows_a1, gsem_a)
    odrain(c_last, out_a, osem_a)
    repack(rows_a0, rows_a1, c_last, out_a)
    ofire(c_last, out_a, osem_a)
    gdrain(idx_b, rows_b0, rows_b1, gsem_b)
    odrain(c_last + 1, out_b, osem_b)
    repack(rows_b0, rows_b1, c_last + 1, out_b)
    ofire(c_last + 1, out_b, osem_b)
    odrain(c_last, out_a, osem_a)
    odrain(c_last + 1, out_b, osem_b)


@jax.jit
def _sc_embed(idx4, token_table, pos_table):
    mesh = plsc.VectorSubcoreMesh(
        core_axis_name="c", subcore_axis_name="s", num_cores=NC, num_subcores=NS
    )
    return pl.kernel(
        _body,
        out_type=jax.ShapeDtypeStruct((L, 8, NW, 8, 128), jnp.float32),
        mesh=mesh,
        scratch_types=[
            pltpu.VMEM((CH_L, BG), jnp.int32),
            pltpu.VMEM((CH_L, BG), jnp.int32),
            pltpu.VMEM((BG, D), jnp.float32),
            pltpu.VMEM((BG, D), jnp.float32),
            pltpu.VMEM((BG, D), jnp.float32),
            pltpu.VMEM((BG, D), jnp.float32),
            pltpu.VMEM((CH_L, 8, 8, 128), jnp.float32),
            pltpu.VMEM((CH_L, 8, 8, 128), jnp.float32),
            pltpu.VMEM((L, D), jnp.float32),
            pltpu.SemaphoreType.DMA,
            pltpu.SemaphoreType.DMA,
            pltpu.SemaphoreType.DMA,
            pltpu.SemaphoreType.DMA,
        ],
        compiler_params=pltpu.CompilerParams(
            use_tc_tiling_on_sc=False, needs_layout_passes=False
        ),
    )(idx4, token_table, pos_table)


def kernel(inputs, token_table, pos_table):
    idx4 = (
        inputs.astype(jnp.int32)
        .reshape(NW, BG, L // 8, 8)
        .transpose(2, 0, 3, 1)
    )
    out5 = _sc_embed(idx4, token_table, pos_table)
    return out5.transpose(2, 4, 0, 1, 3).reshape(B, L, D)


# speedup vs baseline: 2.0557x; 2.0557x over previous
import functools

import jax
import jax.numpy as jnp
from jax import lax
from jax.experimental import pallas as pl
from jax.experimental.pallas import tpu as pltpu
from jax.experimental.pallas import tpu_sc as plsc

B = 4096
L = 200
D = 64
N = B * L
NC = 2
NS = 16
NW = NC * NS
BG = B // NW
CH_L = 2
N_CHUNKS = L // CH_L
LANES = 16


def _body(idx_hbm, tok_hbm, pos_hbm, out_hbm,
          idx_a, idx_b, rows_a0, rows_a1, rows_b0, rows_b1, out_a, out_b,
          pos_v, gsem_a, gsem_b, osem_a, osem_b):
    cid = lax.axis_index("c")
    sid = lax.axis_index("s")
    wid = sid * NC + cid

    pltpu.sync_copy(pos_hbm, pos_v)

    def fire(c, idx_v, rows0, rows1, gsem):
        lt = c >> 2
        ll0 = (c & 3) * CH_L
        pltpu.sync_copy(idx_hbm.at[lt, wid, pl.ds(ll0, CH_L)], idx_v)
        pltpu.async_copy(tok_hbm.at[idx_v.at[0]], rows0, gsem)
        pltpu.async_copy(tok_hbm.at[idx_v.at[1]], rows1, gsem)

    def gdrain(idx_v, rows0, rows1, gsem):
        pltpu.make_async_copy(tok_hbm.at[idx_v.at[0]], rows0, gsem).wait()
        pltpu.make_async_copy(tok_hbm.at[idx_v.at[1]], rows1, gsem).wait()

    def repack(rows0, rows1, c, out_v):
        iota = lax.iota(jnp.int32, LANES)
        dvecs = [iota + (j * LANES) for j in range(D // LANES)]
        i1 = [dv >> 3 for dv in dvecs]
        i2 = [dv & 7 for dv in dvecs]

        def b_body(b, carry):
            bvec = jnp.full((LANES,), b, jnp.int32)
            for p, rows_v in ((0, rows0), (1, rows1)):
                pvec = jnp.full((LANES,), p, jnp.int32)
                l = c * CH_L + p
                for j in range(D // LANES):
                    x = rows_v[b, pl.ds(j * LANES, LANES)] + pos_v[l, pl.ds(j * LANES, LANES)]
                    plsc.store_scatter(out_v, [pvec, i1[j], i2[j], bvec], x)
            return carry

        lax.fori_loop(0, BG, b_body, 0)

    def ofire(c, out_v, osem):
        pltpu.async_copy(
            out_v.at[:, :, :, pl.ds(0, 128)], out_hbm.at[pl.ds(c * CH_L, CH_L), pl.ds(0, 8), wid], osem
        )

    def odrain(c, out_v, osem):
        pltpu.make_async_copy(
            out_v.at[:, :, :, pl.ds(0, 128)], out_hbm.at[pl.ds(c * CH_L, CH_L), pl.ds(0, 8), wid], osem
        ).wait()

    fire(0, idx_a, rows_a0, rows_a1, gsem_a)
    fire(1, idx_b, rows_b0, rows_b1, gsem_b)
    gdrain(idx_a, rows_a0, rows_a1, gsem_a)
    repack(rows_a0, rows_a1, 0, out_a)
    ofire(0, out_a, osem_a)
    fire(2, idx_a, rows_a0, rows_a1, gsem_a)
    gdrain(idx_b, rows_b0, rows_b1, gsem_b)
    repack(rows_b0, rows_b1, 1, out_b)
    ofire(1, out_b, osem_b)
    fire(3, idx_b, rows_b0, rows_b1, gsem_b)

    def pair_body(cc, carry):
        c0 = 2 * cc
        gdrain(idx_a, rows_a0, rows_a1, gsem_a)
        odrain(c0, out_a, osem_a)
        repack(rows_a0, rows_a1, c0, out_a)
        ofire(c0, out_a, osem_a)
        fire(c0 + 2, idx_a, rows_a0, rows_a1, gsem_a)
        gdrain(idx_b, rows_b0, rows_b1, gsem_b)
        odrain(c0 + 1, out_b, osem_b)
        repack(rows_b0, rows_b1, c0 + 1, out_b)
        ofire(c0 + 1, out_b, osem_b)
        fire(c0 + 3, idx_b, rows_b0, rows_b1, gsem_b)
        return carry

    lax.fori_loop(1, N_CHUNKS // 2 - 1, pair_body, 0)

    c_last = N_CHUNKS - 2
    gdrain(idx_a, rows_a0, rows_a1, gsem_a)
    odrain(c_last, out_a, osem_a)
    repack(rows_a0, rows_a1, c_last, out_a)
    ofire(c_last, out_a, osem_a)
    gdrain(idx_b, rows_b0, rows_b1, gsem_b)
    odrain(c_last + 1, out_b, osem_b)
    repack(rows_b0, rows_b1, c_last + 1, out_b)
    ofire(c_last + 1, out_b, osem_b)
    odrain(c_last, out_a, osem_a)
    odrain(c_last + 1, out_b, osem_b)


@jax.jit
def _sc_embed(idx4, token_table, pos_table):
    mesh = plsc.VectorSubcoreMesh(
        core_axis_name="c", subcore_axis_name="s", num_cores=NC, num_subcores=NS
    )
    return pl.kernel(
        _body,
        out_type=jax.ShapeDtypeStruct((L, 8, NW, 8, 128), jnp.float32),
        mesh=mesh,
        scratch_types=[
            pltpu.VMEM((CH_L, BG), jnp.int32),
            pltpu.VMEM((CH_L, BG), jnp.int32),
            pltpu.VMEM((BG, D), jnp.float32),
            pltpu.VMEM((BG, D), jnp.float32),
            pltpu.VMEM((BG, D), jnp.float32),
            pltpu.VMEM((BG, D), jnp.float32),
            pltpu.VMEM((CH_L, 8, 8, 129), jnp.float32),
            pltpu.VMEM((CH_L, 8, 8, 129), jnp.float32),
            pltpu.VMEM((L, D), jnp.float32),
            pltpu.SemaphoreType.DMA,
            pltpu.SemaphoreType.DMA,
            pltpu.SemaphoreType.DMA,
            pltpu.SemaphoreType.DMA,
        ],
        compiler_params=pltpu.CompilerParams(
            use_tc_tiling_on_sc=False, needs_layout_passes=False
        ),
    )(idx4, token_table, pos_table)


def kernel(inputs, token_table, pos_table):
    idx4 = (
        inputs.astype(jnp.int32)
        .reshape(NW, BG, L // 8, 8)
        .transpose(2, 0, 3, 1)
    )
    out5 = _sc_embed(idx4, token_table, pos_table)
    return out5.transpose(2, 4, 0, 1, 3).reshape(B, L, D)


# speedup vs baseline: 3.7773x; 1.8375x over previous
import functools

import jax
import jax.numpy as jnp
from jax import lax
from jax.experimental import pallas as pl
from jax.experimental.pallas import tpu as pltpu
from jax.experimental.pallas import tpu_sc as plsc

B = 4096
L = 200
D = 64
N = B * L
NC = 2
NS = 16
NW = NC * NS
ROWS_PER_W = N // NW
SEQ_PER_W = ROWS_PER_W // L
CH_SEQ = 2
CH_ROWS = CH_SEQ * L
N_CHUNKS = SEQ_PER_W // CH_SEQ
G = 100
N_GATHER = CH_ROWS // G
LANES = 16
DJ = D // LANES


def _body(idx_hbm, tok_hbm, pos_hbm, out_hbm,
          idx_a, idx_b, rows_a, rows_b, out_a, out_b, pos_v,
          gsem_a, gsem_b, osem_a, osem_b):
    cid = lax.axis_index("c")
    sid = lax.axis_index("s")
    wid = sid * NC + cid

    pltpu.sync_copy(pos_hbm, pos_v)

    def fire(c, idx_v, rows_v, gsem):
        idx_row = wid * (ROWS_PER_W // G) + c * N_GATHER
        pltpu.sync_copy(idx_hbm.at[pl.ds(idx_row, N_GATHER)], idx_v)
        for u in range(N_GATHER):
            s, h = divmod(u, L // G)
            pltpu.async_copy(
                tok_hbm.at[idx_v.at[u]],
                rows_v.at[s, pl.ds(h * G, G)],
                gsem,
            )

    def gdrain(idx_v, rows_v, gsem):
        for u in range(N_GATHER):
            s, h = divmod(u, L // G)
            pltpu.make_async_copy(
                tok_hbm.at[idx_v.at[u]],
                rows_v.at[s, pl.ds(h * G, G)],
                gsem,
            ).wait()

    def repack(rows_v, out_v):
        def l_body(lh, carry):
            for par in range(2):
                l = 2 * lh + par
                for j in range(DJ):
                    pv = pos_v[l, pl.ds(j * LANES, LANES)]
                    col = par * D + j * LANES
                    for s in range(CH_SEQ):
                        out_v[s, lh, pl.ds(col, LANES)] = (
                            rows_v[s, l, pl.ds(j * LANES, LANES)] + pv
                        )
            return carry

        lax.fori_loop(0, L // 2, l_body, 0)

    def ofire(c, out_v, osem):
        seq_base = wid * SEQ_PER_W + c * CH_SEQ
        pltpu.async_copy(out_v, out_hbm.at[pl.ds(seq_base, CH_SEQ)], osem)

    def odrain(c, out_v, osem):
        seq_base = wid * SEQ_PER_W + c * CH_SEQ
        pltpu.make_async_copy(
            out_v, out_hbm.at[pl.ds(seq_base, CH_SEQ)], osem
        ).wait()

    fire(0, idx_a, rows_a, gsem_a)
    fire(1, idx_b, rows_b, gsem_b)
    gdrain(idx_a, rows_a, gsem_a)
    repack(rows_a, out_a)
    ofire(0, out_a, osem_a)
    fire(2, idx_a, rows_a, gsem_a)
    gdrain(idx_b, rows_b, gsem_b)
    repack(rows_b, out_b)
    ofire(1, out_b, osem_b)
    fire(3, idx_b, rows_b, gsem_b)

    def pair_body(cc, carry):
        c0 = 2 * cc
        gdrain(idx_a, rows_a, gsem_a)
        odrain(c0, out_a, osem_a)
        repack(rows_a, out_a)
        ofire(c0, out_a, osem_a)
        fire(c0 + 2, idx_a, rows_a, gsem_a)
        gdrain(idx_b, rows_b, gsem_b)
        odrain(c0 + 1, out_b, osem_b)
        repack(rows_b, out_b)
        ofire(c0 + 1, out_b, osem_b)
        fire(c0 + 3, idx_b, rows_b, gsem_b)
        return carry

    lax.fori_loop(1, N_CHUNKS // 2 - 1, pair_body, 0)

    c_last = N_CHUNKS - 2
    gdrain(idx_a, rows_a, gsem_a)
    odrain(c_last, out_a, osem_a)
    repack(rows_a, out_a)
    ofire(c_last, out_a, osem_a)
    gdrain(idx_b, rows_b, gsem_b)
    odrain(c_last + 1, out_b, osem_b)
    repack(rows_b, out_b)
    ofire(c_last + 1, out_b, osem_b)
    odrain(c_last, out_a, osem_a)
    odrain(c_last + 1, out_b, osem_b)


@jax.jit
def _sc_embed(idx2d, token_table, pos_table):
    mesh = plsc.VectorSubcoreMesh(
        core_axis_name="c", subcore_axis_name="s", num_cores=NC, num_subcores=NS
    )
    return pl.kernel(
        _body,
        out_type=jax.ShapeDtypeStruct((B, 104, 128), jnp.float32),
        mesh=mesh,
        scratch_types=[
            pltpu.VMEM((N_GATHER, G), jnp.int32),
            pltpu.VMEM((N_GATHER, G), jnp.int32),
            pltpu.VMEM((CH_SEQ, L, D), jnp.float32),
            pltpu.VMEM((CH_SEQ, L, D), jnp.float32),
            pltpu.VMEM((CH_SEQ, 104, 128), jnp.float32),
            pltpu.VMEM((CH_SEQ, 104, 128), jnp.float32),
            pltpu.VMEM((L, D), jnp.float32),
            pltpu.SemaphoreType.DMA,
            pltpu.SemaphoreType.DMA,
            pltpu.SemaphoreType.DMA,
            pltpu.SemaphoreType.DMA,
        ],
        compiler_params=pltpu.CompilerParams(use_tc_tiling_on_sc=False),
    )(idx2d, token_table, pos_table)


def kernel(inputs, token_table, pos_table):
    idx2d = inputs.reshape(N // G, G).astype(jnp.int32)
    out = _sc_embed(idx2d, token_table, pos_table)
    return out[:, : L // 2, :].reshape(B, L, D)


# speedup vs baseline: 3.8019x; 1.0065x over previous
import functools

import jax
import jax.numpy as jnp
from jax import lax
from jax.experimental import pallas as pl
from jax.experimental.pallas import tpu as pltpu
from jax.experimental.pallas import tpu_sc as plsc

B = 4096
L = 200
D = 64
N = B * L
NC = 2
NS = 16
NW = NC * NS
ROWS_PER_W = N // NW
SEQ_PER_W = ROWS_PER_W // L
CH_SEQ = 2
CH_ROWS = CH_SEQ * L
N_CHUNKS = SEQ_PER_W // CH_SEQ
G = 100
N_GATHER = CH_ROWS // G
LANES = 16
DJ = D // LANES


def _body(idx_hbm, tok_hbm, pos_hbm, out_hbm,
          idx_a, idx_b, rows_a, rows_b, out_a, out_b, pos_v,
          gsem_a, gsem_b, osem_a, osem_b):
    cid = lax.axis_index("c")
    sid = lax.axis_index("s")
    wid = sid * NC + cid

    pltpu.sync_copy(pos_hbm, pos_v)

    def fire(c, idx_v, rows_v, gsem):
        idx_row = wid * (ROWS_PER_W // G) + c * N_GATHER
        pltpu.sync_copy(idx_hbm.at[pl.ds(idx_row, N_GATHER)], idx_v)
        for u in range(N_GATHER):
            s, h = divmod(u, L // G)
            pltpu.async_copy(
                tok_hbm.at[idx_v.at[u]],
                rows_v.at[s, pl.ds(h * G, G)],
                gsem,
            )

    def gdrain(idx_v, rows_v, gsem):
        for u in range(N_GATHER):
            s, h = divmod(u, L // G)
            pltpu.make_async_copy(
                tok_hbm.at[idx_v.at[u]],
                rows_v.at[s, pl.ds(h * G, G)],
                gsem,
            ).wait()

    def repack(rows_v, out_v):
        @plsc.parallel_loop(0, L // 2, unroll=2)
        def l_body(lh):
            for par in range(2):
                l = 2 * lh + par
                for j in range(DJ):
                    pv = pos_v[l, pl.ds(j * LANES, LANES)]
                    col = par * D + j * LANES
                    for s in range(CH_SEQ):
                        out_v[s, lh, pl.ds(col, LANES)] = (
                            rows_v[s, l, pl.ds(j * LANES, LANES)] + pv
                        )

    def ofire(c, out_v, osem):
        seq_base = wid * SEQ_PER_W + c * CH_SEQ
        pltpu.async_copy(out_v, out_hbm.at[pl.ds(seq_base, CH_SEQ)], osem)

    def odrain(c, out_v, osem):
        seq_base = wid * SEQ_PER_W + c * CH_SEQ
        pltpu.make_async_copy(
            out_v, out_hbm.at[pl.ds(seq_base, CH_SEQ)], osem
        ).wait()

    fire(0, idx_a, rows_a, gsem_a)
    fire(1, idx_b, rows_b, gsem_b)
    gdrain(idx_a, rows_a, gsem_a)
    repack(rows_a, out_a)
    ofire(0, out_a, osem_a)
    fire(2, idx_a, rows_a, gsem_a)
    gdrain(idx_b, rows_b, gsem_b)
    repack(rows_b, out_b)
    ofire(1, out_b, osem_b)
    fire(3, idx_b, rows_b, gsem_b)

    def pair_body(cc, carry):
        c0 = 2 * cc
        gdrain(idx_a, rows_a, gsem_a)
        odrain(c0, out_a, osem_a)
        repack(rows_a, out_a)
        ofire(c0, out_a, osem_a)
        fire(c0 + 2, idx_a, rows_a, gsem_a)
        gdrain(idx_b, rows_b, gsem_b)
        odrain(c0 + 1, out_b, osem_b)
        repack(rows_b, out_b)
        ofire(c0 + 1, out_b, osem_b)
        fire(c0 + 3, idx_b, rows_b, gsem_b)
        return carry

    lax.fori_loop(1, N_CHUNKS // 2 - 1, pair_body, 0)

    c_last = N_CHUNKS - 2
    gdrain(idx_a, rows_a, gsem_a)
    odrain(c_last, out_a, osem_a)
    repack(rows_a, out_a)
    ofire(c_last, out_a, osem_a)
    gdrain(idx_b, rows_b, gsem_b)
    odrain(c_last + 1, out_b, osem_b)
    repack(rows_b, out_b)
    ofire(c_last + 1, out_b, osem_b)
    odrain(c_last, out_a, osem_a)
    odrain(c_last + 1, out_b, osem_b)


@jax.jit
def _sc_embed(idx2d, token_table, pos_table):
    mesh = plsc.VectorSubcoreMesh(
        core_axis_name="c", subcore_axis_name="s", num_cores=NC, num_subcores=NS
    )
    return pl.kernel(
        _body,
        out_type=jax.ShapeDtypeStruct((B, 104, 128), jnp.float32),
        mesh=mesh,
        scratch_types=[
            pltpu.VMEM((N_GATHER, G), jnp.int32),
            pltpu.VMEM((N_GATHER, G), jnp.int32),
            pltpu.VMEM((CH_SEQ, L, D), jnp.float32),
            pltpu.VMEM((CH_SEQ, L, D), jnp.float32),
            pltpu.VMEM((CH_SEQ, 104, 128), jnp.float32),
            pltpu.VMEM((CH_SEQ, 104, 128), jnp.float32),
            pltpu.VMEM((L, D), jnp.float32),
            pltpu.SemaphoreType.DMA,
            pltpu.SemaphoreType.DMA,
            pltpu.SemaphoreType.DMA,
            pltpu.SemaphoreType.DMA,
        ],
        compiler_params=pltpu.CompilerParams(use_tc_tiling_on_sc=False),
    )(idx2d, token_table, pos_table)


def kernel(inputs, token_table, pos_table):
    idx2d = inputs.reshape(N // G, G).astype(jnp.int32)
    out = _sc_embed(idx2d, token_table, pos_table)
    return out[:, : L // 2, :].reshape(B, L, D)
